# baseline (speedup 1.0000x reference)
import functools
import math

import jax
import jax.numpy as jnp
from jax import lax
from jax.experimental import pallas as pl
from jax.experimental.pallas import tpu as pltpu
from jax.experimental.pallas import tpu_sc as plsc

N = 10000
E = 160000
F = 128
CUTOFF = 5.0
NRBF = 20
NRBF_PAD = 32

BN = 1000
BE = 4096

NC = 2
NS = 16
NW = NC * NS
GCH = 128
GNCH = 40
EPW = GCH * GNCH
EP = NW * EPW
NPAIR = GNCH // 2
SN = 632
N_ACC = NS * SN


def _silu(x):
    return x * jax.nn.sigmoid(x)


def _pack2(a, b):
    au = lax.bitcast_convert_type(a, jnp.uint32) + jnp.uint32(0x8000)
    bu = lax.bitcast_convert_type(b, jnp.uint32) + jnp.uint32(0x8000)
    w = (au >> 16) | ((bu >> 16) << 16)
    return lax.bitcast_convert_type(w, jnp.int32)


def _unpack_lo(u):
    w = lax.bitcast_convert_type(u, jnp.uint32)
    return lax.bitcast_convert_type(w << 16, jnp.float32)


def _unpack_hi(u):
    w = lax.bitcast_convert_type(u, jnp.uint32)
    return lax.bitcast_convert_type((w >> 16) << 16, jnp.float32)


def _emb_body(s_ref, w_ref, b_ref, o_ref):
    o_ref[...] = (
        jnp.dot(s_ref[...], w_ref[...], preferred_element_type=jnp.float32)
        + b_ref[...]
    )


def _emb(s, w, b):
    return pl.pallas_call(
        _emb_body,
        grid=(N // BN,),
        in_specs=[
            pl.BlockSpec((BN, F), lambda i: (i, 0)),
            pl.BlockSpec((F, F), lambda i: (0, 0)),
            pl.BlockSpec((1, F), lambda i: (0, 0)),
        ],
        out_specs=pl.BlockSpec((BN, F), lambda i: (i, 0)),
        out_shape=jax.ShapeDtypeStruct((N, F), jnp.float32),
    )(s, w, b.reshape(1, F))


def _phi_body(s_ref, w1_ref, b1_ref, w2_ref, b2_ref, o_ref):
    x = jnp.dot(s_ref[...], w1_ref[...], preferred_element_type=jnp.float32)
    x = _silu(x + b1_ref[...])
    r = (
        jnp.dot(x, w2_ref[...], preferred_element_type=jnp.float32)
        + b2_ref[...]
    )
    o_ref[:, :F] = _pack2(r[:, :F], r[:, 2 * F:])
    o_ref[:, F:] = _pack2(r[:, F:2 * F], jnp.zeros_like(r[:, :F]))


def _phi(s, w1, b1, w2, b2):
    return pl.pallas_call(
        _phi_body,
        grid=(N // BN,),
        in_specs=[
            pl.BlockSpec((BN, F), lambda i: (i, 0)),
            pl.BlockSpec((F, F), lambda i: (0, 0)),
            pl.BlockSpec((1, F), lambda i: (0, 0)),
            pl.BlockSpec((F, 3 * F), lambda i: (0, 0)),
            pl.BlockSpec((1, 3 * F), lambda i: (0, 0)),
        ],
        out_specs=pl.BlockSpec((BN, 2 * F), lambda i: (i, 0)),
        out_shape=jax.ShapeDtypeStruct((N, 2 * F), jnp.int32),
    )(s, w1, b1.reshape(1, F), w2, b2.reshape(1, 3 * F))


def _attr_body(ps_ref, pd_ref, o_ref):
    o_ref[...] = ps_ref[:, :16] - pd_ref[:, :16]


def _attr(ps128, pd128):
    return pl.pallas_call(
        _attr_body,
        grid=(EP // BE,),
        in_specs=[
            pl.BlockSpec((BE, 128), lambda i: (i, 0)),
            pl.BlockSpec((BE, 128), lambda i: (i, 0)),
        ],
        out_specs=pl.BlockSpec((BE, 16), lambda i: (i, 0)),
        out_shape=jax.ShapeDtypeStruct((EP, 16), jnp.float32),
    )(ps128, pd128)


def _msg_body(has_v, phi_ref, vj_ref, attr_ref, fr_ref, rw_ref, rb_ref,
              os_ref, o0_ref, o1_ref, o2_ref):
    a = attr_ref[...]
    r = jnp.sqrt(jnp.sum(a * a, axis=1, keepdims=True) + 1e-12)
    ax = r * fr_ref[...]
    rbf = jnp.sin(ax) / r
    ch1 = (
        jnp.dot(rbf, rw_ref[...], preferred_element_type=jnp.float32)
        + rb_ref[...]
    )
    cut = 0.5 * (jnp.cos(r * (math.pi / CUTOFF)) + 1.0)
    cut = cut * (r < CUTOFF).astype(jnp.float32)
    u = phi_ref[...]
    gl = ch1[:, :F] * cut
    gm = ch1[:, F:2 * F] * cut
    gr = ch1[:, 2 * F:] * cut
    left = _unpack_lo(u[:, :F]) * gl
    right = _unpack_hi(u[:, :F]) * gr
    dsm = _unpack_lo(u[:, F:]) * gm
    rinv = 1.0 / r
    os_ref[...] = dsm
    outs = (o0_ref, o1_ref, o2_ref)
    for c in range(3):
        nc = a[:, c:c + 1] * rinv
        dv = right * nc
        if has_v:
            uv = vj_ref[:, (c % 2) * F:(c % 2) * F + F]
            vjc = _unpack_lo(uv) if c < 2 else _unpack_hi(uv)
            dv = dv + vjc * left
        outs[c][...] = dv


def _msg(phi_j, v_j, attr, freqs, rw, rb, has_v):
    body = functools.partial(_msg_body, has_v)
    shp = jax.ShapeDtypeStruct((EP, F), jnp.float32)
    if has_v:
        vspec = pl.BlockSpec((BE, 2 * F), lambda i: (i, 0))
    else:
        vspec = pl.BlockSpec((8, 2 * F), lambda i: (0, 0))
    return pl.pallas_call(
        body,
        grid=(EP // BE,),
        in_specs=[
            pl.BlockSpec((BE, 2 * F), lambda i: (i, 0)),
            vspec,
            pl.BlockSpec((BE, 16), lambda i: (i, 0)),
            pl.BlockSpec((1, NRBF_PAD), lambda i: (0, 0)),
            pl.BlockSpec((NRBF_PAD, 3 * F), lambda i: (0, 0)),
            pl.BlockSpec((1, 3 * F), lambda i: (0, 0)),
        ],
        out_specs=[pl.BlockSpec((BE, F), lambda i: (i, 0))] * 4,
        out_shape=[shp, shp, shp, shp],
    )(phi_j, v_j, attr, freqs, rw, rb)


def _upd_body(s_ref, v_ref, agg_ref, du_ref, dv_ref, up_ref, upb_ref,
              l2_ref, l2b_ref, os_ref, ov_ref, ovb_ref):
    agg = agg_ref[...]
    s1 = s_ref[...] + agg[0, 0] + agg[1, 0]
    v1 = [v_ref[:, c * F:(c + 1) * F] + agg[0, c + 1] + agg[1, c + 1]
          for c in range(3)]
    U = [jnp.dot(v1[c], du_ref[...], preferred_element_type=jnp.float32)
         for c in range(3)]
    V = [jnp.dot(v1[c], dv_ref[...], preferred_element_type=jnp.float32)
         for c in range(3)]
    UV = U[0] * V[0] + U[1] * V[1] + U[2] * V[2]
    nV = jnp.sqrt(V[0] * V[0] + V[1] * V[1] + V[2] * V[2] + 1e-12)
    h = (
        jnp.dot(s1, up_ref[:F], preferred_element_type=jnp.float32)
        + jnp.dot(nV, up_ref[F:], preferred_element_type=jnp.float32)
        + upb_ref[...]
    )
    h = _silu(h)
    su = (
        jnp.dot(h, l2_ref[...], preferred_element_type=jnp.float32)
        + l2b_ref[...]
    )
    top = su[:, :F]
    mid = su[:, F:2 * F]
    bot = su[:, 2 * F:]
    os_ref[...] = s1 + mid * UV + bot
    vn = [v1[c] * (1.0 + top) for c in range(3)]
    for c in range(3):
        ov_ref[:, c * F:(c + 1) * F] = vn[c]
    ovb_ref[:, :F] = _pack2(vn[0], vn[2])
    ovb_ref[:, F:] = _pack2(vn[1], jnp.zeros_like(vn[0]))


def _upd(s, v3f, agg2, du, dvw, up, upb, l2, l2b):
    return pl.pallas_call(
        _upd_body,
        grid=(N // BN,),
        in_specs=[
            pl.BlockSpec((BN, F), lambda i: (i, 0)),
            pl.BlockSpec((BN, 3 * F), lambda i: (i, 0)),
            pl.BlockSpec((2, 4, BN, F), lambda i: (0, 0, i, 0)),
            pl.BlockSpec((F, F), lambda i: (0, 0)),
            pl.BlockSpec((F, F), lambda i: (0, 0)),
            pl.BlockSpec((2 * F, F), lambda i: (0, 0)),
            pl.BlockSpec((1, F), lambda i: (0, 0)),
            pl.BlockSpec((F, 3 * F), lambda i: (0, 0)),
            pl.BlockSpec((1, 3 * F), lambda i: (0, 0)),
        ],
        out_specs=[
            pl.BlockSpec((BN, F), lambda i: (i, 0)),
            pl.BlockSpec((BN, 3 * F), lambda i: (i, 0)),
            pl.BlockSpec((BN, 2 * F), lambda i: (i, 0)),
        ],
        out_shape=[
            jax.ShapeDtypeStruct((N, F), jnp.float32),
            jax.ShapeDtypeStruct((N, 3 * F), jnp.float32),
            jax.ShapeDtypeStruct((N, 2 * F), jnp.int32),
        ],
    )(s, v3f, agg2, du, dvw, up, upb.reshape(1, F), l2, l2b.reshape(1, 3 * F))


def _out_body(s_ref, v_ref, v1a_ref, v2a_ref, u0a_ref, u0ab_ref, u2a_ref,
              u2ab_ref, v1b_ref, pc_ref, u0b_ref, u0bb_ref, u2b_ref,
              u2bb_ref, o_ref):
    H = F // 2
    v = [v_ref[:, c * F:(c + 1) * F] for c in range(3)]
    t = [jnp.dot(v[c], v1a_ref[...], preferred_element_type=jnp.float32)
         for c in range(3)]
    vec1 = jnp.sqrt(t[0] * t[0] + t[1] * t[1] + t[2] * t[2] + 1e-12)
    vec2 = [jnp.dot(v[c], v2a_ref[...], preferred_element_type=jnp.float32)
            for c in range(3)]
    h = (
        jnp.dot(s_ref[...], u0a_ref[:F], preferred_element_type=jnp.float32)
        + jnp.dot(vec1, u0a_ref[F:], preferred_element_type=jnp.float32)
        + u0ab_ref[...]
    )
    h = _silu(h) * (1.0 / 0.6)
    h = (
        jnp.dot(h, u2a_ref[...], preferred_element_type=jnp.float32)
        + u2ab_ref[...]
    )
    x1 = _silu(h[:, :H])
    gate = h[:, H:]
    w = [gate * vec2[c] for c in range(3)]
    t2 = [jnp.dot(w[c], v1b_ref[...], preferred_element_type=jnp.float32)
          for c in range(3)]
    vec1b = jnp.sqrt(t2[0] * t2[0] + t2[1] * t2[1] + t2[2] * t2[2] + 1e-12)
    hb = (
        jnp.dot(x1, u0b_ref[:H], preferred_element_type=jnp.float32)
        + jnp.dot(vec1b, u0b_ref[H:], preferred_element_type=jnp.float32)
        + u0bb_ref[...]
    )
    hb = _silu(hb) * (1.0 / 0.6)
    h2b = (
        jnp.dot(hb, u2b_ref[...], preferred_element_type=jnp.float32)
        + u2bb_ref[...]
    )
    gate2 = h2b[:, 1:2]
    acc = (
        jnp.dot(w[0], pc_ref[:H], preferred_element_type=jnp.float32)
        + jnp.dot(w[1], pc_ref[H:2 * H], preferred_element_type=jnp.float32)
        + jnp.dot(w[2], pc_ref[2 * H:], preferred_element_type=jnp.float32)
    )
    o_ref[...] = acc * gate2


def _out_heads(s, v3f, p0, p1):
    H = F // 2
    u2b = jnp.zeros((H, 128), jnp.float32).at[:, :2].set(p1['u2_W'])
    u2bb = jnp.zeros((1, 128), jnp.float32).at[0, :2].set(p1['u2_b'])
    pc = jnp.zeros((3 * H, 128), jnp.float32)
    for c in range(3):
        pc = pc.at[c * H:(c + 1) * H, c].set(p1['v2_W'][:, 0])
    return pl.pallas_call(
        _out_body,
        grid=(N // BN,),
        in_specs=[
            pl.BlockSpec((BN, F), lambda i: (i, 0)),
            pl.BlockSpec((BN, 3 * F), lambda i: (i, 0)),
            pl.BlockSpec((F, F), lambda i: (0, 0)),
            pl.BlockSpec((F, H), lambda i: (0, 0)),
            pl.BlockSpec((2 * F, F), lambda i: (0, 0)),
            pl.BlockSpec((1, F), lambda i: (0, 0)),
            pl.BlockSpec((F, 2 * H), lambda i: (0, 0)),
            pl.BlockSpec((1, 2 * H), lambda i: (0, 0)),
            pl.BlockSpec((H, H), lambda i: (0, 0)),
            pl.BlockSpec((3 * H, 128), lambda i: (0, 0)),
            pl.BlockSpec((2 * H, H), lambda i: (0, 0)),
            pl.BlockSpec((1, H), lambda i: (0, 0)),
            pl.BlockSpec((H, 128), lambda i: (0, 0)),
            pl.BlockSpec((1, 128), lambda i: (0, 0)),
        ],
        out_specs=pl.BlockSpec((BN, 128), lambda i: (i, 0)),
        out_shape=jax.ShapeDtypeStruct((N, 128), jnp.float32),
    )(s, v3f, p0['v1_W'], p0['v2_W'], p0['u0_W'], p0['u0_b'].reshape(1, F),
      p0['u2_W'], p0['u2_b'].reshape(1, 2 * H), p1['v1_W'], pc, p1['u0_W'],
      p1['u0_b'].reshape(1, H), u2b, u2bb)


def _sc_gather(table, idx3):
    D = table.shape[1]
    dt = table.dtype
    mesh = plsc.VectorSubcoreMesh(core_axis_name="c", subcore_axis_name="s")

    @functools.partial(
        pl.kernel, mesh=mesh,
        out_type=jax.ShapeDtypeStruct((EP, D), dt),
        scratch_types=[
            pltpu.VMEM((GNCH, GCH), jnp.int32),
            pltpu.VMEM((GCH, D), dt),
            pltpu.VMEM((GCH, D), dt),
            pltpu.SemaphoreType.DMA,
            pltpu.SemaphoreType.DMA,
        ],
    )
    def k(table_hbm, idx_hbm, out_hbm, idx_v, b0, b1, s0, s1):
        wid = lax.axis_index("s") * NC + lax.axis_index("c")
        base = wid * EPW
        pltpu.sync_copy(idx_hbm.at[wid], idx_v)
        pltpu.async_copy(table_hbm.at[idx_v.at[0]], b0, s0)

        def body(jj, _):
            j0 = 2 * jj
            pltpu.async_copy(table_hbm.at[idx_v.at[j0 + 1]], b1, s1)
            pltpu.make_async_copy(table_hbm.at[idx_v.at[j0]], b0, s0).wait()
            pltpu.sync_copy(b0, out_hbm.at[pl.ds(base + j0 * GCH, GCH)])

            @pl.when(jj + 1 < NPAIR)
            def _():
                pltpu.async_copy(table_hbm.at[idx_v.at[j0 + 2]], b0, s0)

            pltpu.make_async_copy(table_hbm.at[idx_v.at[j0 + 1]], b1, s1).wait()
            pltpu.sync_copy(b1, out_hbm.at[pl.ds(base + (j0 + 1) * GCH, GCH)])
            return 0

        lax.fori_loop(0, NPAIR, body, 0)

    return k(table, idx3)


def _sc_gather2(tab_a, idx_a, tab_b, idx_b):
    D = tab_a.shape[1]
    dt = tab_a.dtype
    mesh = plsc.VectorSubcoreMesh(core_axis_name="c", subcore_axis_name="s")

    @functools.partial(
        pl.kernel, mesh=mesh,
        out_type=(jax.ShapeDtypeStruct((EP, D), dt),
                  jax.ShapeDtypeStruct((EP, D), dt)),
        scratch_types=[
            pltpu.VMEM((GNCH, GCH), jnp.int32),
            pltpu.VMEM((GCH, D), dt),
            pltpu.VMEM((GCH, D), dt),
            pltpu.SemaphoreType.DMA,
            pltpu.SemaphoreType.DMA,
        ],
    )
    def k(ta_h, ia_h, tb_h, ib_h, oa_h, ob_h, idx_v, b0, b1, s0, s1):
        wid = lax.axis_index("s") * NC + lax.axis_index("c")
        base = wid * EPW
        for t_h, i_h, o_h in ((ta_h, ia_h, oa_h), (tb_h, ib_h, ob_h)):
            pltpu.sync_copy(i_h.at[wid], idx_v)
            pltpu.async_copy(t_h.at[idx_v.at[0]], b0, s0)

            def body(jj, _, t_h=t_h, o_h=o_h):
                j0 = 2 * jj
                pltpu.async_copy(t_h.at[idx_v.at[j0 + 1]], b1, s1)
                pltpu.make_async_copy(t_h.at[idx_v.at[j0]], b0, s0).wait()
                pltpu.sync_copy(b0, o_h.at[pl.ds(base + j0 * GCH, GCH)])

                @pl.when(jj + 1 < NPAIR)
                def _():
                    pltpu.async_copy(t_h.at[idx_v.at[j0 + 2]], b0, s0)

                pltpu.make_async_copy(t_h.at[idx_v.at[j0 + 1]], b1, s1).wait()
                pltpu.sync_copy(b1, o_h.at[pl.ds(base + (j0 + 1) * GCH, GCH)])
                return 0

            lax.fori_loop(0, NPAIR, body, 0)

    return k(tab_a, idx_a, tab_b, idx_b)


def _sc_scatter(msgs, dst3, zrow):
    mesh = plsc.VectorSubcoreMesh(core_axis_name="c", subcore_axis_name="s")

    @functools.partial(
        pl.kernel, mesh=mesh,
        out_type=jax.ShapeDtypeStruct((2, 4, N_ACC, F), jnp.float32),
        scratch_types=[
            pltpu.VMEM((GNCH, GCH), jnp.int32),
            pltpu.VMEM((GCH, F), jnp.float32),
            pltpu.VMEM((GCH, F), jnp.float32),
            pltpu.VMEM_SHARED((N_ACC, F), jnp.float32),
            pltpu.SemaphoreType.DMA,
            pltpu.SemaphoreType.DMA,
        ],
    )
    def k(m0h, m1h, m2h, m3h, dsth, zh, outh, idx_v, b0, b1, acc, s0, s1):
        cid = lax.axis_index("c")
        sid = lax.axis_index("s")
        wid = sid * NC + cid
        base = wid * EPW
        pltpu.sync_copy(dsth.at[wid], idx_v)
        for m, mh in enumerate((m0h, m1h, m2h, m3h)):
            pltpu.sync_copy(zh, acc.at[pl.ds(sid * SN, SN)])
            plsc.subcore_barrier()
            pltpu.async_copy(mh.at[pl.ds(base, GCH)], b0, s0)

            def body(jj, _):
                j0 = 2 * jj
                pltpu.async_copy(mh.at[pl.ds(base + (j0 + 1) * GCH, GCH)],
                                 b1, s1)
                pltpu.make_async_copy(mh.at[pl.ds(base + j0 * GCH, GCH)],
                                      b0, s0).wait()
                pltpu.sync_copy(b0, acc.at[idx_v.at[j0]], add=True)

                @pl.when(jj + 1 < NPAIR)
                def _():
                    pltpu.async_copy(mh.at[pl.ds(base + (j0 + 2) * GCH, GCH)],
                                     b0, s0)

                pltpu.make_async_copy(mh.at[pl.ds(base + (j0 + 1) * GCH, GCH)],
                                      b1, s1).wait()
                pltpu.sync_copy(b1, acc.at[idx_v.at[j0 + 1]], add=True)
                return 0

            lax.fori_loop(0, NPAIR, body, 0)
            plsc.subcore_barrier()
            pltpu.sync_copy(acc.at[pl.ds(sid * SN, SN)],
                            outh.at[cid, m, pl.ds(sid * SN, SN)])

    return k(msgs[0], msgs[1], msgs[2], msgs[3], dst3, zrow)


def kernel(s, pos, params, edge_index):
    npad = EPW - E // NW
    src2 = edge_index[0].astype(jnp.int32).reshape(NW, E // NW)
    dst2 = edge_index[1].astype(jnp.int32).reshape(NW, E // NW)
    gpad = jnp.arange(npad, dtype=jnp.int32) * (N // (npad + 1))
    spad = N + (jnp.arange(npad, dtype=jnp.int32) % (N_ACC - N))
    src3 = jnp.concatenate(
        [src2, jnp.broadcast_to(gpad, (NW, npad))], axis=1
    ).reshape(NW, GNCH, GCH)
    dst3 = jnp.concatenate(
        [dst2, jnp.broadcast_to(spad, (NW, npad))], axis=1
    ).reshape(NW, GNCH, GCH)
    pos_pad = jnp.zeros((N, 128), jnp.float32).at[:, :3].set(pos)
    ps128, pd128 = _sc_gather2(pos_pad, src3, pos_pad, dst3)
    attr = _attr(ps128, pd128)
    zrow = jnp.zeros((SN, F), jnp.float32)
    v_dummy = jnp.zeros((8, 2 * F), jnp.int32)

    freqs = jnp.zeros((1, NRBF_PAD), jnp.float32).at[0, :NRBF].set(
        jnp.arange(1, NRBF + 1, dtype=jnp.float32) * (math.pi / CUTOFF))

    s = _emb(s, params['emb_W'], params['emb_b'])
    v3f = jnp.zeros((N, 3 * F), jnp.float32)

    for li, lp in enumerate(params['layers']):
        rw = jnp.zeros((NRBF_PAD, 3 * F), jnp.float32).at[:NRBF].set(
            lp['m_rbf_W'])
        phi = _phi(s, lp['m_lin1_W'], lp['m_lin1_b'],
                   lp['m_lin2_W'], lp['m_lin2_b'])
        if li == 0:
            phi_j = _sc_gather(phi, src3)
            v_j = v_dummy
        else:
            phi_j, v_j = _sc_gather2(phi, src3, v_bf, src3)
        msgs = _msg(phi_j, v_j, attr, freqs, rw,
                    lp['m_rbf_b'].reshape(1, -1), has_v=(li != 0))
        agg2 = _sc_scatter(msgs, dst3, zrow)
        s, v3f, v_bf = _upd(s, v3f, agg2, lp['u_dU_W'], lp['u_dV_W'],
                            lp['u_up_W'], lp['u_up_b'], lp['u_lin2_W'],
                            lp['u_lin2_b'])

    out = _out_heads(s, v3f, params['out'][0], params['out'][1])
    return out[:, :3]


# speedup vs baseline: 1.0053x; 1.0053x over previous
import functools
import math

import jax
import jax.numpy as jnp
from jax import lax
from jax.experimental import pallas as pl
from jax.experimental.pallas import tpu as pltpu
from jax.experimental.pallas import tpu_sc as plsc

N = 10000
E = 160000
F = 128
CUTOFF = 5.0
NRBF = 20
NRBF_PAD = 32

BN = 1000
BE = 2048

NC = 2
NS = 16
NW = NC * NS
GCH = 128
GNCH = 40
EPW = GCH * GNCH
EP = NW * EPW
NPAIR = GNCH // 2
SN = 632
N_ACC = NS * SN


def _silu(x):
    return x * jax.nn.sigmoid(x)


def _pack2(a, b):
    au = lax.bitcast_convert_type(a, jnp.uint32) + jnp.uint32(0x8000)
    bu = lax.bitcast_convert_type(b, jnp.uint32) + jnp.uint32(0x8000)
    w = (au >> 16) | ((bu >> 16) << 16)
    return lax.bitcast_convert_type(w, jnp.int32)


def _unpack_lo(u):
    w = lax.bitcast_convert_type(u, jnp.uint32)
    return lax.bitcast_convert_type(w << 16, jnp.float32)


def _unpack_hi(u):
    w = lax.bitcast_convert_type(u, jnp.uint32)
    return lax.bitcast_convert_type((w >> 16) << 16, jnp.float32)


def _emb_body(s_ref, w_ref, b_ref, o_ref):
    o_ref[...] = (
        jnp.dot(s_ref[...], w_ref[...], preferred_element_type=jnp.float32)
        + b_ref[...]
    )


def _emb(s, w, b):
    return pl.pallas_call(
        _emb_body,
        grid=(N // BN,),
        in_specs=[
            pl.BlockSpec((BN, F), lambda i: (i, 0)),
            pl.BlockSpec((F, F), lambda i: (0, 0)),
            pl.BlockSpec((1, F), lambda i: (0, 0)),
        ],
        out_specs=pl.BlockSpec((BN, F), lambda i: (i, 0)),
        out_shape=jax.ShapeDtypeStruct((N, F), jnp.float32),
    )(s, w, b.reshape(1, F))


def _phi_body(s_ref, w1_ref, b1_ref, w2_ref, b2_ref, o_ref):
    x = jnp.dot(s_ref[...], w1_ref[...], preferred_element_type=jnp.float32)
    x = _silu(x + b1_ref[...])
    r = (
        jnp.dot(x, w2_ref[...], preferred_element_type=jnp.float32)
        + b2_ref[...]
    )
    o_ref[:, :F] = _pack2(r[:, :F], r[:, 2 * F:])
    o_ref[:, F:] = _pack2(r[:, F:2 * F], jnp.zeros_like(r[:, :F]))


def _phi(s, w1, b1, w2, b2):
    return pl.pallas_call(
        _phi_body,
        grid=(N // BN,),
        in_specs=[
            pl.BlockSpec((BN, F), lambda i: (i, 0)),
            pl.BlockSpec((F, F), lambda i: (0, 0)),
            pl.BlockSpec((1, F), lambda i: (0, 0)),
            pl.BlockSpec((F, 3 * F), lambda i: (0, 0)),
            pl.BlockSpec((1, 3 * F), lambda i: (0, 0)),
        ],
        out_specs=pl.BlockSpec((BN, 2 * F), lambda i: (i, 0)),
        out_shape=jax.ShapeDtypeStruct((N, 2 * F), jnp.int32),
    )(s, w1, b1.reshape(1, F), w2, b2.reshape(1, 3 * F))


def _attr_body(ps_ref, pd_ref, o_ref):
    o_ref[...] = ps_ref[:, :16] - pd_ref[:, :16]


def _attr(ps128, pd128):
    return pl.pallas_call(
        _attr_body,
        grid=(EP // BE,),
        in_specs=[
            pl.BlockSpec((BE, 128), lambda i: (i, 0)),
            pl.BlockSpec((BE, 128), lambda i: (i, 0)),
        ],
        out_specs=pl.BlockSpec((BE, 16), lambda i: (i, 0)),
        out_shape=jax.ShapeDtypeStruct((EP, 16), jnp.float32),
    )(ps128, pd128)


def _msg_body(has_v, phi_ref, vj_ref, attr_ref, fr_ref, rw_ref, rb_ref,
              os_ref, o0_ref, o1_ref, o2_ref):
    a = attr_ref[...]
    r = jnp.sqrt(jnp.sum(a * a, axis=1, keepdims=True) + 1e-12)
    ax = r * fr_ref[...]
    rbf = jnp.sin(ax) / r
    ch1 = (
        jnp.dot(rbf, rw_ref[...], preferred_element_type=jnp.float32)
        + rb_ref[...]
    )
    cut = 0.5 * (jnp.cos(r * (math.pi / CUTOFF)) + 1.0)
    cut = cut * (r < CUTOFF).astype(jnp.float32)
    u = phi_ref[...]
    gl = ch1[:, :F] * cut
    gm = ch1[:, F:2 * F] * cut
    gr = ch1[:, 2 * F:] * cut
    left = _unpack_lo(u[:, :F]) * gl
    right = _unpack_hi(u[:, :F]) * gr
    dsm = _unpack_lo(u[:, F:]) * gm
    rinv = 1.0 / r
    os_ref[...] = dsm
    outs = (o0_ref, o1_ref, o2_ref)
    for c in range(3):
        nc = a[:, c:c + 1] * rinv
        dv = right * nc
        if has_v:
            uv = vj_ref[:, (c % 2) * F:(c % 2) * F + F]
            vjc = _unpack_lo(uv) if c < 2 else _unpack_hi(uv)
            dv = dv + vjc * left
        outs[c][...] = dv


def _msg(phi_j, v_j, attr, freqs, rw, rb, has_v):
    body = functools.partial(_msg_body, has_v)
    shp = jax.ShapeDtypeStruct((EP, F), jnp.float32)
    if has_v:
        vspec = pl.BlockSpec((BE, 2 * F), lambda i: (i, 0))
    else:
        vspec = pl.BlockSpec((8, 2 * F), lambda i: (0, 0))
    return pl.pallas_call(
        body,
        grid=(EP // BE,),
        in_specs=[
            pl.BlockSpec((BE, 2 * F), lambda i: (i, 0)),
            vspec,
            pl.BlockSpec((BE, 16), lambda i: (i, 0)),
            pl.BlockSpec((1, NRBF_PAD), lambda i: (0, 0)),
            pl.BlockSpec((NRBF_PAD, 3 * F), lambda i: (0, 0)),
            pl.BlockSpec((1, 3 * F), lambda i: (0, 0)),
        ],
        out_specs=[pl.BlockSpec((BE, F), lambda i: (i, 0))] * 4,
        out_shape=[shp, shp, shp, shp],
    )(phi_j, v_j, attr, freqs, rw, rb)


def _upd_body(s_ref, v_ref, agg_ref, du_ref, dv_ref, up_ref, upb_ref,
              l2_ref, l2b_ref, os_ref, ov_ref, ovb_ref):
    agg = agg_ref[...]
    s1 = s_ref[...] + agg[0, 0] + agg[1, 0]
    v1 = [v_ref[:, c * F:(c + 1) * F] + agg[0, c + 1] + agg[1, c + 1]
          for c in range(3)]
    U = [jnp.dot(v1[c], du_ref[...], preferred_element_type=jnp.float32)
         for c in range(3)]
    V = [jnp.dot(v1[c], dv_ref[...], preferred_element_type=jnp.float32)
         for c in range(3)]
    UV = U[0] * V[0] + U[1] * V[1] + U[2] * V[2]
    nV = jnp.sqrt(V[0] * V[0] + V[1] * V[1] + V[2] * V[2] + 1e-12)
    h = (
        jnp.dot(s1, up_ref[:F], preferred_element_type=jnp.float32)
        + jnp.dot(nV, up_ref[F:], preferred_element_type=jnp.float32)
        + upb_ref[...]
    )
    h = _silu(h)
    su = (
        jnp.dot(h, l2_ref[...], preferred_element_type=jnp.float32)
        + l2b_ref[...]
    )
    top = su[:, :F]
    mid = su[:, F:2 * F]
    bot = su[:, 2 * F:]
    os_ref[...] = s1 + mid * UV + bot
    vn = [v1[c] * (1.0 + top) for c in range(3)]
    for c in range(3):
        ov_ref[:, c * F:(c + 1) * F] = vn[c]
    ovb_ref[:, :F] = _pack2(vn[0], vn[2])
    ovb_ref[:, F:] = _pack2(vn[1], jnp.zeros_like(vn[0]))


def _upd(s, v3f, agg2, du, dvw, up, upb, l2, l2b):
    return pl.pallas_call(
        _upd_body,
        grid=(N // BN,),
        in_specs=[
            pl.BlockSpec((BN, F), lambda i: (i, 0)),
            pl.BlockSpec((BN, 3 * F), lambda i: (i, 0)),
            pl.BlockSpec((2, 4, BN, F), lambda i: (0, 0, i, 0)),
            pl.BlockSpec((F, F), lambda i: (0, 0)),
            pl.BlockSpec((F, F), lambda i: (0, 0)),
            pl.BlockSpec((2 * F, F), lambda i: (0, 0)),
            pl.BlockSpec((1, F), lambda i: (0, 0)),
            pl.BlockSpec((F, 3 * F), lambda i: (0, 0)),
            pl.BlockSpec((1, 3 * F), lambda i: (0, 0)),
        ],
        out_specs=[
            pl.BlockSpec((BN, F), lambda i: (i, 0)),
            pl.BlockSpec((BN, 3 * F), lambda i: (i, 0)),
            pl.BlockSpec((BN, 2 * F), lambda i: (i, 0)),
        ],
        out_shape=[
            jax.ShapeDtypeStruct((N, F), jnp.float32),
            jax.ShapeDtypeStruct((N, 3 * F), jnp.float32),
            jax.ShapeDtypeStruct((N, 2 * F), jnp.int32),
        ],
    )(s, v3f, agg2, du, dvw, up, upb.reshape(1, F), l2, l2b.reshape(1, 3 * F))


def _out_body(s_ref, v_ref, v1a_ref, v2a_ref, u0a_ref, u0ab_ref, u2a_ref,
              u2ab_ref, v1b_ref, pc_ref, u0b_ref, u0bb_ref, u2b_ref,
              u2bb_ref, o_ref):
    H = F // 2
    v = [v_ref[:, c * F:(c + 1) * F] for c in range(3)]
    t = [jnp.dot(v[c], v1a_ref[...], preferred_element_type=jnp.float32)
         for c in range(3)]
    vec1 = jnp.sqrt(t[0] * t[0] + t[1] * t[1] + t[2] * t[2] + 1e-12)
    vec2 = [jnp.dot(v[c], v2a_ref[...], preferred_element_type=jnp.float32)
            for c in range(3)]
    h = (
        jnp.dot(s_ref[...], u0a_ref[:F], preferred_element_type=jnp.float32)
        + jnp.dot(vec1, u0a_ref[F:], preferred_element_type=jnp.float32)
        + u0ab_ref[...]
    )
    h = _silu(h) * (1.0 / 0.6)
    h = (
        jnp.dot(h, u2a_ref[...], preferred_element_type=jnp.float32)
        + u2ab_ref[...]
    )
    x1 = _silu(h[:, :H])
    gate = h[:, H:]
    w = [gate * vec2[c] for c in range(3)]
    t2 = [jnp.dot(w[c], v1b_ref[...], preferred_element_type=jnp.float32)
          for c in range(3)]
    vec1b = jnp.sqrt(t2[0] * t2[0] + t2[1] * t2[1] + t2[2] * t2[2] + 1e-12)
    hb = (
        jnp.dot(x1, u0b_ref[:H], preferred_element_type=jnp.float32)
        + jnp.dot(vec1b, u0b_ref[H:], preferred_element_type=jnp.float32)
        + u0bb_ref[...]
    )
    hb = _silu(hb) * (1.0 / 0.6)
    h2b = (
        jnp.dot(hb, u2b_ref[...], preferred_element_type=jnp.float32)
        + u2bb_ref[...]
    )
    gate2 = h2b[:, 1:2]
    acc = (
        jnp.dot(w[0], pc_ref[:H], preferred_element_type=jnp.float32)
        + jnp.dot(w[1], pc_ref[H:2 * H], preferred_element_type=jnp.float32)
        + jnp.dot(w[2], pc_ref[2 * H:], preferred_element_type=jnp.float32)
    )
    o_ref[...] = acc * gate2


def _out_heads(s, v3f, p0, p1):
    H = F // 2
    u2b = jnp.zeros((H, 128), jnp.float32).at[:, :2].set(p1['u2_W'])
    u2bb = jnp.zeros((1, 128), jnp.float32).at[0, :2].set(p1['u2_b'])
    pc = jnp.zeros((3 * H, 128), jnp.float32)
    for c in range(3):
        pc = pc.at[c * H:(c + 1) * H, c].set(p1['v2_W'][:, 0])
    return pl.pallas_call(
        _out_body,
        grid=(N // BN,),
        in_specs=[
            pl.BlockSpec((BN, F), lambda i: (i, 0)),
            pl.BlockSpec((BN, 3 * F), lambda i: (i, 0)),
            pl.BlockSpec((F, F), lambda i: (0, 0)),
            pl.BlockSpec((F, H), lambda i: (0, 0)),
            pl.BlockSpec((2 * F, F), lambda i: (0, 0)),
            pl.BlockSpec((1, F), lambda i: (0, 0)),
            pl.BlockSpec((F, 2 * H), lambda i: (0, 0)),
            pl.BlockSpec((1, 2 * H), lambda i: (0, 0)),
            pl.BlockSpec((H, H), lambda i: (0, 0)),
            pl.BlockSpec((3 * H, 128), lambda i: (0, 0)),
            pl.BlockSpec((2 * H, H), lambda i: (0, 0)),
            pl.BlockSpec((1, H), lambda i: (0, 0)),
            pl.BlockSpec((H, 128), lambda i: (0, 0)),
            pl.BlockSpec((1, 128), lambda i: (0, 0)),
        ],
        out_specs=pl.BlockSpec((BN, 128), lambda i: (i, 0)),
        out_shape=jax.ShapeDtypeStruct((N, 128), jnp.float32),
    )(s, v3f, p0['v1_W'], p0['v2_W'], p0['u0_W'], p0['u0_b'].reshape(1, F),
      p0['u2_W'], p0['u2_b'].reshape(1, 2 * H), p1['v1_W'], pc, p1['u0_W'],
      p1['u0_b'].reshape(1, H), u2b, u2bb)


def _sc_gather(table, idx3):
    D = table.shape[1]
    dt = table.dtype
    mesh = plsc.VectorSubcoreMesh(core_axis_name="c", subcore_axis_name="s")

    @functools.partial(
        pl.kernel, mesh=mesh,
        out_type=jax.ShapeDtypeStruct((EP, D), dt),
        scratch_types=[
            pltpu.VMEM((GNCH, GCH), jnp.int32),
            pltpu.VMEM((GCH, D), dt),
            pltpu.VMEM((GCH, D), dt),
            pltpu.SemaphoreType.DMA,
            pltpu.SemaphoreType.DMA,
        ],
    )
    def k(table_hbm, idx_hbm, out_hbm, idx_v, b0, b1, s0, s1):
        wid = lax.axis_index("s") * NC + lax.axis_index("c")
        base = wid * EPW
        pltpu.sync_copy(idx_hbm.at[wid], idx_v)
        pltpu.async_copy(table_hbm.at[idx_v.at[0]], b0, s0)

        def body(jj, _):
            j0 = 2 * jj
            pltpu.async_copy(table_hbm.at[idx_v.at[j0 + 1]], b1, s1)
            pltpu.make_async_copy(table_hbm.at[idx_v.at[j0]], b0, s0).wait()
            pltpu.sync_copy(b0, out_hbm.at[pl.ds(base + j0 * GCH, GCH)])

            @pl.when(jj + 1 < NPAIR)
            def _():
                pltpu.async_copy(table_hbm.at[idx_v.at[j0 + 2]], b0, s0)

            pltpu.make_async_copy(table_hbm.at[idx_v.at[j0 + 1]], b1, s1).wait()
            pltpu.sync_copy(b1, out_hbm.at[pl.ds(base + (j0 + 1) * GCH, GCH)])
            return 0

        lax.fori_loop(0, NPAIR, body, 0)

    return k(table, idx3)


def _sc_gather2(tab_a, idx_a, tab_b, idx_b):
    D = tab_a.shape[1]
    dt = tab_a.dtype
    mesh = plsc.VectorSubcoreMesh(core_axis_name="c", subcore_axis_name="s")

    @functools.partial(
        pl.kernel, mesh=mesh,
        out_type=(jax.ShapeDtypeStruct((EP, D), dt),
                  jax.ShapeDtypeStruct((EP, D), dt)),
        scratch_types=[
            pltpu.VMEM((GNCH, GCH), jnp.int32),
            pltpu.VMEM((GCH, D), dt),
            pltpu.VMEM((GCH, D), dt),
            pltpu.SemaphoreType.DMA,
            pltpu.SemaphoreType.DMA,
        ],
    )
    def k(ta_h, ia_h, tb_h, ib_h, oa_h, ob_h, idx_v, b0, b1, s0, s1):
        wid = lax.axis_index("s") * NC + lax.axis_index("c")
        base = wid * EPW
        for t_h, i_h, o_h in ((ta_h, ia_h, oa_h), (tb_h, ib_h, ob_h)):
            pltpu.sync_copy(i_h.at[wid], idx_v)
            pltpu.async_copy(t_h.at[idx_v.at[0]], b0, s0)

            def body(jj, _, t_h=t_h, o_h=o_h):
                j0 = 2 * jj
                pltpu.async_copy(t_h.at[idx_v.at[j0 + 1]], b1, s1)
                pltpu.make_async_copy(t_h.at[idx_v.at[j0]], b0, s0).wait()
                pltpu.sync_copy(b0, o_h.at[pl.ds(base + j0 * GCH, GCH)])

                @pl.when(jj + 1 < NPAIR)
                def _():
                    pltpu.async_copy(t_h.at[idx_v.at[j0 + 2]], b0, s0)

                pltpu.make_async_copy(t_h.at[idx_v.at[j0 + 1]], b1, s1).wait()
                pltpu.sync_copy(b1, o_h.at[pl.ds(base + (j0 + 1) * GCH, GCH)])
                return 0

            lax.fori_loop(0, NPAIR, body, 0)

    return k(tab_a, idx_a, tab_b, idx_b)


def _sc_scatter(msgs, dst3, zrow):
    mesh = plsc.VectorSubcoreMesh(core_axis_name="c", subcore_axis_name="s")

    @functools.partial(
        pl.kernel, mesh=mesh,
        out_type=jax.ShapeDtypeStruct((2, 4, N_ACC, F), jnp.float32),
        scratch_types=[
            pltpu.VMEM((GNCH, GCH), jnp.int32),
            pltpu.VMEM((GCH, F), jnp.float32),
            pltpu.VMEM((GCH, F), jnp.float32),
            pltpu.VMEM_SHARED((N_ACC, F), jnp.float32),
            pltpu.SemaphoreType.DMA,
            pltpu.SemaphoreType.DMA,
        ],
    )
    def k(m0h, m1h, m2h, m3h, dsth, zh, outh, idx_v, b0, b1, acc, s0, s1):
        cid = lax.axis_index("c")
        sid = lax.axis_index("s")
        wid = sid * NC + cid
        base = wid * EPW
        pltpu.sync_copy(dsth.at[wid], idx_v)
        for m, mh in enumerate((m0h, m1h, m2h, m3h)):
            pltpu.sync_copy(zh, acc.at[pl.ds(sid * SN, SN)])
            plsc.subcore_barrier()
            pltpu.async_copy(mh.at[pl.ds(base, GCH)], b0, s0)

            def body(jj, _):
                j0 = 2 * jj
                pltpu.async_copy(mh.at[pl.ds(base + (j0 + 1) * GCH, GCH)],
                                 b1, s1)
                pltpu.make_async_copy(mh.at[pl.ds(base + j0 * GCH, GCH)],
                                      b0, s0).wait()
                pltpu.sync_copy(b0, acc.at[idx_v.at[j0]], add=True)

                @pl.when(jj + 1 < NPAIR)
                def _():
                    pltpu.async_copy(mh.at[pl.ds(base + (j0 + 2) * GCH, GCH)],
                                     b0, s0)

                pltpu.make_async_copy(mh.at[pl.ds(base + (j0 + 1) * GCH, GCH)],
                                      b1, s1).wait()
                pltpu.sync_copy(b1, acc.at[idx_v.at[j0 + 1]], add=True)
                return 0

            lax.fori_loop(0, NPAIR, body, 0)
            plsc.subcore_barrier()
            pltpu.sync_copy(acc.at[pl.ds(sid * SN, SN)],
                            outh.at[cid, m, pl.ds(sid * SN, SN)])

    return k(msgs[0], msgs[1], msgs[2], msgs[3], dst3, zrow)


def kernel(s, pos, params, edge_index):
    npad = EPW - E // NW
    src2 = edge_index[0].astype(jnp.int32).reshape(NW, E // NW)
    dst2 = edge_index[1].astype(jnp.int32).reshape(NW, E // NW)
    gpad = jnp.arange(npad, dtype=jnp.int32) * (N // (npad + 1))
    spad = N + (jnp.arange(npad, dtype=jnp.int32) % (N_ACC - N))
    src3 = jnp.concatenate(
        [src2, jnp.broadcast_to(gpad, (NW, npad))], axis=1
    ).reshape(NW, GNCH, GCH)
    dst3 = jnp.concatenate(
        [dst2, jnp.broadcast_to(spad, (NW, npad))], axis=1
    ).reshape(NW, GNCH, GCH)
    pos_pad = jnp.zeros((N, 128), jnp.float32).at[:, :3].set(pos)
    ps128, pd128 = _sc_gather2(pos_pad, src3, pos_pad, dst3)
    attr = _attr(ps128, pd128)
    zrow = jnp.zeros((SN, F), jnp.float32)
    v_dummy = jnp.zeros((8, 2 * F), jnp.int32)

    freqs = jnp.zeros((1, NRBF_PAD), jnp.float32).at[0, :NRBF].set(
        jnp.arange(1, NRBF + 1, dtype=jnp.float32) * (math.pi / CUTOFF))

    s = _emb(s, params['emb_W'], params['emb_b'])
    v3f = jnp.zeros((N, 3 * F), jnp.float32)

    for li, lp in enumerate(params['layers']):
        rw = jnp.zeros((NRBF_PAD, 3 * F), jnp.float32).at[:NRBF].set(
            lp['m_rbf_W'])
        phi = _phi(s, lp['m_lin1_W'], lp['m_lin1_b'],
                   lp['m_lin2_W'], lp['m_lin2_b'])
        if li == 0:
            phi_j = _sc_gather(phi, src3)
            v_j = v_dummy
        else:
            phi_j, v_j = _sc_gather2(phi, src3, v_bf, src3)
        msgs = _msg(phi_j, v_j, attr, freqs, rw,
                    lp['m_rbf_b'].reshape(1, -1), has_v=(li != 0))
        agg2 = _sc_scatter(msgs, dst3, zrow)
        s, v3f, v_bf = _upd(s, v3f, agg2, lp['u_dU_W'], lp['u_dV_W'],
                            lp['u_up_W'], lp['u_up_b'], lp['u_lin2_W'],
                            lp['u_lin2_b'])

    out = _out_heads(s, v3f, params['out'][0], params['out'][1])
    return out[:, :3]
